# baseline (speedup 1.0000x reference)
import functools

import jax
import jax.numpy as jnp
from jax import lax
from jax.experimental import pallas as pl
from jax.experimental.pallas import tpu as pltpu
from jax.experimental.pallas import tpu_sc as plsc

_BN = 128
_K0 = 2


def _build_masks(parents_index):
    Cn, cc, p = parents_index.shape
    if p == 0:
        return jnp.zeros((Cn, 128), jnp.int32)
    valid = parents_index < cc
    q = jnp.where(valid, parents_index, 0)
    sh = (q & 31).astype(jnp.uint32)
    val = jnp.left_shift(jnp.uint32(1), sh)
    zero = jnp.uint32(0)
    lo = jnp.where(valid & (q < 32), val, zero)
    hi = jnp.where(valid & (q >= 32), val, zero)
    lo = lax.reduce(lo, zero, lax.bitwise_or, (2,))
    hi = lax.reduce(hi, zero, lax.bitwise_or, (2,))
    masks = jnp.concatenate([lo, hi], axis=1)
    masks = lax.bitcast_convert_type(masks, jnp.int32)
    return jnp.pad(masks, [(0, 0), (0, 128 - masks.shape[1])])


def _build_masks16(masks, c):
    words = masks[:, :2 * c]
    low = (words & 0xFFFF).astype(jnp.float32)
    high = lax.shift_right_logical(words, 16).astype(jnp.float32)
    return jnp.concatenate(
        [low[:, :c], high[:, :c], low[:, c:], high[:, c:]], axis=1)


def _sc_gather_rows(table, idx):
    info = plsc.get_sparse_core_info()
    nc, ns = info.num_cores, info.num_subcores
    nw = nc * ns
    B = idx.shape[0]
    D = table.shape[1]
    b_per_w = B // nw
    mesh = plsc.VectorSubcoreMesh(core_axis_name="c", subcore_axis_name="s")

    @functools.partial(
        pl.kernel,
        mesh=mesh,
        out_type=jax.ShapeDtypeStruct((B, D), jnp.int32),
        scratch_types=[
            pltpu.VMEM((b_per_w,), jnp.int32),
            pltpu.VMEM((b_per_w, D), jnp.int32),
            pltpu.SemaphoreType.DMA,
        ],
    )
    def gather_kernel(table_hbm, idx_hbm, out_hbm, idx_v, rows_v, sem):
        wid = lax.axis_index("s") * nc + lax.axis_index("c")
        base = wid * b_per_w
        pltpu.sync_copy(idx_hbm.at[pl.ds(base, b_per_w)], idx_v)
        pltpu.async_copy(table_hbm.at[idx_v], rows_v, sem).wait()
        pltpu.sync_copy(rows_v, out_hbm.at[pl.ds(base, b_per_w)])

    return gather_kernel(table, idx)


def _bitmask_const(q):
    m = 1 << (q % 32)
    return jnp.int32(m - 2**32 if m >= 2**31 else m)


def _core(wf_ref, lo, hi, owf_ref, optp_ref, c):
    bn = wf_ref.shape[0]
    rows_max = []
    rows_min = []
    for n in range(bn):
        slab = wf_ref[n]
        rows_max.append(jnp.max(slab, axis=0, keepdims=True))
        rows_min.append(jnp.min(slab, axis=0, keepdims=True))
    wmax = jnp.concatenate(rows_max, axis=0)
    wmin = jnp.concatenate(rows_min, axis=0)
    ptp = wmax - wmin

    big = jnp.float32(1e30)
    zero = jnp.int32(0)
    pmin = jnp.full((bn, c), big, jnp.float32)
    for q in range(c):
        word = lo if q < 32 else hi
        hit = (word & _bitmask_const(q)) != zero
        vq = ptp[:, q:q + 1]
        pmin = jnp.minimum(pmin, jnp.where(hit, vq, big))
    resc = jnp.minimum(pmin / ptp, jnp.float32(1.0))

    optp_ref[...] = ptp * resc
    owf_ref[...] = wf_ref[...] * resc[:, None, :]


def _tc_body_mxu(wf_ref, mc_ref, mf_ref, owf_ref, optp_ref, *, c):
    mcb = mc_ref[0]
    nv = mf_ref.shape[0]
    iota = lax.broadcasted_iota(jnp.int32, (1, nv), 1)
    onehot = (mcb == iota).astype(jnp.float32)
    rows = jnp.dot(onehot, mf_ref[...],
                   preferred_element_type=jnp.float32)
    ll = rows[:, 0:c].astype(jnp.int32)
    lh = rows[:, c:2 * c].astype(jnp.int32)
    hl = rows[:, 2 * c:3 * c].astype(jnp.int32)
    hh = rows[:, 3 * c:4 * c].astype(jnp.int32)
    lo = ll | (lh << 16)
    hi = hl | (hh << 16)
    _core(wf_ref, lo, hi, owf_ref, optp_ref, c)


def _tc_body_sc(wf_ref, pm_ref, _owf_in, _optp_in, owf_ref, optp_ref, *, c):
    _core(wf_ref, pm_ref[:, :c], pm_ref[:, c:2 * c], owf_ref, optp_ref, c)


def kernel(waveforms, max_channels, parents_index):
    N, T, c = waveforms.shape
    bn = _BN
    nb = N // bn
    k0 = _K0
    masks = _build_masks(parents_index)
    masksf = _build_masks16(masks, c)
    pim = _sc_gather_rows(masks, max_channels[k0 * bn:])
    mc3 = max_channels[:k0 * bn].reshape(k0, bn, 1)

    out_shape = [
        jax.ShapeDtypeStruct((N, T, c), jnp.float32),
        jax.ShapeDtypeStruct((N, c), jnp.float32),
    ]
    params = pltpu.CompilerParams(dimension_semantics=("parallel",))

    part_wf, part_ptp = pl.pallas_call(
        functools.partial(_tc_body_mxu, c=c),
        grid=(k0,),
        in_specs=[
            pl.BlockSpec((bn, T, c), lambda i: (i, 0, 0)),
            pl.BlockSpec((1, bn, 1), lambda i: (i, 0, 0)),
            pl.BlockSpec(masksf.shape, lambda i: (0, 0)),
        ],
        out_specs=[
            pl.BlockSpec((bn, T, c), lambda i: (i, 0, 0)),
            pl.BlockSpec((bn, c), lambda i: (i, 0)),
        ],
        out_shape=out_shape,
        compiler_params=params,
    )(waveforms, mc3, masksf)

    out_wf, out_ptp = pl.pallas_call(
        functools.partial(_tc_body_sc, c=c),
        grid=(nb - k0,),
        in_specs=[
            pl.BlockSpec((bn, T, c), lambda i: (i + k0, 0, 0)),
            pl.BlockSpec((bn, 128), lambda i: (i, 0)),
            pl.BlockSpec(memory_space=pl.ANY),
            pl.BlockSpec(memory_space=pl.ANY),
        ],
        out_specs=[
            pl.BlockSpec((bn, T, c), lambda i: (i + k0, 0, 0)),
            pl.BlockSpec((bn, c), lambda i: (i + k0, 0)),
        ],
        out_shape=out_shape,
        input_output_aliases={2: 0, 3: 1},
        compiler_params=params,
    )(waveforms, pim, part_wf, part_ptp)
    return out_wf, out_ptp


# speedup vs baseline: 1.0096x; 1.0096x over previous
import functools

import jax
import jax.numpy as jnp
from jax import lax
from jax.experimental import pallas as pl
from jax.experimental.pallas import tpu as pltpu
from jax.experimental.pallas import tpu_sc as plsc

_BN = 128


def _build_masks(parents_index):
    Cn, cc, p = parents_index.shape
    if p == 0:
        return jnp.zeros((Cn, 128), jnp.int32)
    valid = parents_index < cc
    q = jnp.where(valid, parents_index, 0)
    sh = (q & 31).astype(jnp.uint32)
    val = jnp.left_shift(jnp.uint32(1), sh)
    zero = jnp.uint32(0)
    lo = jnp.where(valid & (q < 32), val, zero)
    hi = jnp.where(valid & (q >= 32), val, zero)
    lo = lax.reduce(lo, zero, lax.bitwise_or, (2,))
    hi = lax.reduce(hi, zero, lax.bitwise_or, (2,))
    masks = jnp.concatenate([lo, hi], axis=1)
    masks = lax.bitcast_convert_type(masks, jnp.int32)
    return jnp.pad(masks, [(0, 0), (0, 128 - masks.shape[1])])


def _sc_gather_rows(table, idx):
    info = plsc.get_sparse_core_info()
    nc, ns = info.num_cores, info.num_subcores
    nw = nc * ns
    B = idx.shape[0]
    D = table.shape[1]
    b_per_w = B // nw
    mesh = plsc.VectorSubcoreMesh(core_axis_name="c", subcore_axis_name="s")

    @functools.partial(
        pl.kernel,
        mesh=mesh,
        out_type=jax.ShapeDtypeStruct((B, D), jnp.int32),
        scratch_types=[
            pltpu.VMEM((b_per_w,), jnp.int32),
            pltpu.VMEM((b_per_w, D), jnp.int32),
            pltpu.SemaphoreType.DMA,
        ],
    )
    def gather_kernel(table_hbm, idx_hbm, out_hbm, idx_v, rows_v, sem):
        wid = lax.axis_index("s") * nc + lax.axis_index("c")
        base = wid * b_per_w
        pltpu.sync_copy(idx_hbm.at[pl.ds(base, b_per_w)], idx_v)
        pltpu.async_copy(table_hbm.at[idx_v], rows_v, sem).wait()
        pltpu.sync_copy(rows_v, out_hbm.at[pl.ds(base, b_per_w)])

    return gather_kernel(table, idx)


def _bitmask_const(q):
    m = 1 << (q % 32)
    return jnp.int32(m - 2**32 if m >= 2**31 else m)


def _tc_body(wf_ref, pm_ref, owf_ref, optp_ref, *, c, T):
    bn = wf_ref.shape[0]
    rows_max = []
    rows_min = []
    for n in range(bn):
        slab = wf_ref[n]
        rows_max.append(jnp.max(slab, axis=0, keepdims=True))
        rows_min.append(jnp.min(slab, axis=0, keepdims=True))
    wmax = jnp.concatenate(rows_max, axis=0)
    wmin = jnp.concatenate(rows_min, axis=0)
    ptp = wmax - wmin

    big = jnp.float32(1e30)
    zero = jnp.int32(0)
    lo = pm_ref[:, :c]
    hi = pm_ref[:, c:2 * c]
    pmin = jnp.full((bn, c), big, jnp.float32)
    for q in range(c):
        word = lo if q < 32 else hi
        hit = (word & _bitmask_const(q)) != zero
        vq = ptp[:, q:q + 1]
        pmin = jnp.minimum(pmin, jnp.where(hit, vq, big))
    resc = jnp.minimum(pmin / ptp, jnp.float32(1.0))

    optp_ref[...] = ptp * resc
    owf_ref[...] = wf_ref[...] * resc[:, None, :]


def kernel(waveforms, max_channels, parents_index):
    N, T, c = waveforms.shape
    masks = _build_masks(parents_index)
    pim = _sc_gather_rows(masks, max_channels)
    bn = _BN
    out_wf, out_ptp = pl.pallas_call(
        functools.partial(_tc_body, c=c, T=T),
        grid=(N // bn,),
        in_specs=[
            pl.BlockSpec((bn, T, c), lambda i: (i, 0, 0)),
            pl.BlockSpec((bn, 128), lambda i: (i, 0)),
        ],
        out_specs=[
            pl.BlockSpec((bn, T, c), lambda i: (i, 0, 0)),
            pl.BlockSpec((bn, c), lambda i: (i, 0)),
        ],
        out_shape=[
            jax.ShapeDtypeStruct((N, T, c), jnp.float32),
            jax.ShapeDtypeStruct((N, c), jnp.float32),
        ],
        compiler_params=pltpu.CompilerParams(
            dimension_semantics=("parallel",),
        ),
    )(waveforms, pim)
    return out_wf, out_ptp
